# baseline (speedup 1.0000x reference)
import functools

import jax
import jax.numpy as jnp
from jax import lax
from jax.experimental import pallas as pl
from jax.experimental.pallas import tpu as pltpu
from jax.experimental.pallas import tpu_sc as plsc

NC = 2
NS = 16
NW = NC * NS

IDX_W = 128
L = 16


def _lookup_kernel(B0, B1, V, D):
    W = B0 // NW
    G = W // IDX_W
    mesh = plsc.VectorSubcoreMesh(core_axis_name="c", subcore_axis_name="s")

    @functools.partial(
        pl.kernel,
        out_type=jax.ShapeDtypeStruct((B1, D, B0), jnp.float32),
        mesh=mesh,
        scratch_types=[
            pltpu.VMEM((2, W), jnp.int32),
            pltpu.VMEM((2, W, D), jnp.float32),
            pltpu.VMEM((2, D, W + L), jnp.float32),
            pltpu.SemaphoreType.DMA,
            pltpu.SemaphoreType.DMA,
            pltpu.SemaphoreType.DMA,
            pltpu.SemaphoreType.DMA,
        ],
        compiler_params=pltpu.CompilerParams(
            use_tc_tiling_on_sc=False, needs_layout_passes=False),
    )
    def body(x_hbm, table_hbm, out_hbm, ibuf, rows_v, tbuf, gs0, gs1, os0,
             os1):
        gsem = (gs0, gs1)
        osem = (os0, os1)
        wid = lax.axis_index("s") * NC + lax.axis_index("c")
        b0_0 = wid * W
        lane = lax.iota(jnp.int32, L)

        def fire(c, p):
            pltpu.sync_copy(x_hbm.at[pl.ds(c * B0 + b0_0, W)], ibuf.at[p])
            for j in range(G):
                pltpu.async_copy(
                    table_hbm.at[ibuf.at[p].at[pl.ds(j * IDX_W, IDX_W)]],
                    rows_v.at[p].at[pl.ds(j * IDX_W, IDX_W), :],
                    gsem[p],
                )

        def drain_rows(p):
            pltpu.make_async_copy(
                table_hbm.at[pl.ds(0, W), :], rows_v.at[p], gsem[p]).wait()

        def wait_out(p):
            pltpu.make_async_copy(
                out_hbm.at[0, :, pl.ds(0, W)],
                tbuf.at[p].at[:, pl.ds(0, W)], osem[p]).wait()

        def transpose_store(c, p):
            rows = rows_v.at[p]
            tb = tbuf.at[p]

            dlane = [lane + dh * L for dh in range(D // L)]

            @plsc.parallel_loop(0, W, unroll=4)
            def tp(r):
                rvec = jnp.full((L,), 0, jnp.int32) + r
                for dh in range(D // L):
                    v = rows[r, pl.ds(dh * L, L)]
                    plsc.store_scatter(tb, [dlane[dh], rvec], v)

            pltpu.async_copy(
                tb.at[:, pl.ds(0, W)], out_hbm.at[c, :, pl.ds(b0_0, W)],
                osem[p])

        for p in range(2):
            pltpu.async_copy(
                tbuf.at[p].at[:, pl.ds(0, W)],
                out_hbm.at[p, :, pl.ds(b0_0, W)], osem[p])
        fire(0, 0)

        def outer(t, carry):
            for sub in range(2):
                i = 2 * t + 1 + sub
                p = (1 + sub) % 2
                q = 1 - p
                c_fire = lax.select(i < B1, i, B1 - 1)
                fire(c_fire, p)
                drain_rows(q)
                wait_out(q)
                transpose_store(i - 1, q)
            return carry

        lax.fori_loop(0, B1 // 2, outer, 0)

        drain_rows(0)
        for p in range(2):
            wait_out(p)

    return body


def kernel(x, table):
    B0, B1 = x.shape
    V, D = table.shape
    xt_flat = jnp.transpose(x).reshape(-1).astype(jnp.int32)
    out = _lookup_kernel(B0, B1, V, D)(xt_flat, table)
    return jnp.transpose(out, (2, 0, 1))


# speedup vs baseline: 1.0592x; 1.0592x over previous
import functools

import jax
import jax.numpy as jnp
from jax import lax
from jax.experimental import pallas as pl
from jax.experimental.pallas import tpu as pltpu
from jax.experimental.pallas import tpu_sc as plsc

NC = 2
NS = 16
NW = NC * NS

IDX_W = 128
L = 16


def _lookup_kernel(B0, B1, V, D):
    W = B0 // NW
    G = W // IDX_W
    mesh = plsc.VectorSubcoreMesh(core_axis_name="c", subcore_axis_name="s")

    @functools.partial(
        pl.kernel,
        out_type=jax.ShapeDtypeStruct((B1, D, B0), jnp.float32),
        mesh=mesh,
        scratch_types=[
            pltpu.VMEM((2, W), jnp.int32),
            pltpu.VMEM((2, W, D), jnp.float32),
            pltpu.VMEM((2, D, W + L), jnp.float32),
            pltpu.SemaphoreType.DMA,
            pltpu.SemaphoreType.DMA,
            pltpu.SemaphoreType.DMA,
            pltpu.SemaphoreType.DMA,
        ],
        compiler_params=pltpu.CompilerParams(
            use_tc_tiling_on_sc=False, needs_layout_passes=False),
    )
    def body(x_hbm, table_hbm, out_hbm, ibuf, rows_v, tbuf, gs0, gs1, os0,
             os1):
        gsem = (gs0, gs1)
        osem = (os0, os1)
        wid = lax.axis_index("s") * NC + lax.axis_index("c")
        b0_0 = wid * W
        lane = lax.iota(jnp.int32, L)

        def fire(c, p):
            pltpu.sync_copy(x_hbm.at[pl.ds(c * B0 + b0_0, W)], ibuf.at[p])
            for j in range(G):
                pltpu.async_copy(
                    table_hbm.at[ibuf.at[p].at[pl.ds(j * IDX_W, IDX_W)]],
                    rows_v.at[p].at[pl.ds(j * IDX_W, IDX_W), :],
                    gsem[p],
                )

        def drain_rows(p):
            pltpu.make_async_copy(
                table_hbm.at[pl.ds(0, W), :], rows_v.at[p], gsem[p]).wait()

        def wait_out(p):
            pltpu.make_async_copy(
                out_hbm.at[0, :, pl.ds(0, W)],
                tbuf.at[p].at[:, pl.ds(0, W)], osem[p]).wait()

        def transpose_store(c, p):
            rows = rows_v.at[p]
            tb = tbuf.at[p]

            dlane = [lane + dh * L for dh in range(D // L)]

            @plsc.parallel_loop(0, W, unroll=4)
            def tp(r):
                rvec = jnp.full((L,), 0, jnp.int32) + r
                for dh in range(D // L):
                    v = rows[r, pl.ds(dh * L, L)]
                    plsc.store_scatter(tb, [dlane[dh], rvec], v)

            pltpu.async_copy(
                tb.at[:, pl.ds(0, W)], out_hbm.at[c, :, pl.ds(b0_0, W)],
                osem[p])

        for p in range(2):
            pltpu.async_copy(
                tbuf.at[p].at[:, pl.ds(0, W)],
                out_hbm.at[p, :, pl.ds(b0_0, W)], osem[p])
        fire(0, 0)

        def outer(t, carry):
            for sub in range(2):
                i = 2 * t + 1 + sub
                p = (1 + sub) % 2
                q = 1 - p
                c_fire = lax.select(i < B1, i, B1 - 1)
                fire(c_fire, p)
                drain_rows(q)
                wait_out(q)
                transpose_store(i - 1, q)
            return carry

        lax.fori_loop(0, B1 // 2, outer, 0)

        drain_rows(0)
        for p in range(2):
            wait_out(p)

    return body


def _table_transpose_kernel(V, D):
    GFULL = V // IDX_W
    TAIL = V - GFULL * IDX_W
    NGI = 2 * ((GFULL // NW + 2) // 2)
    mesh = plsc.VectorSubcoreMesh(core_axis_name="c", subcore_axis_name="s")

    @functools.partial(
        pl.kernel,
        out_type=jax.ShapeDtypeStruct((V * D // IDX_W, IDX_W), jnp.float32),
        mesh=mesh,
        scratch_types=[
            pltpu.VMEM((2, D, IDX_W + 1), jnp.float32),
            pltpu.VMEM((2, D, IDX_W), jnp.float32),
            pltpu.SemaphoreType.DMA,
            pltpu.SemaphoreType.DMA,
            pltpu.SemaphoreType.DMA,
            pltpu.SemaphoreType.DMA,
        ],
        compiler_params=pltpu.CompilerParams(needs_layout_passes=False),
    )
    def body(tt_hbm, out_hbm, inbuf, obuf, is0, is1, os0, os1):
        isem = (is0, is1)
        osem = (os0, os1)
        wid = lax.axis_index("s") * NC + lax.axis_index("c")
        lane = lax.iota(jnp.int32, L)
        dlane = [lane + dh * L for dh in range(D // L)]

        def grp(i):
            g = i * NW + wid
            return lax.select(g < GFULL, g, GFULL - 1)

        def fire_in(i, p):
            pltpu.async_copy(
                tt_hbm.at[:, pl.ds(grp(i) * IDX_W, IDX_W)],
                inbuf.at[p].at[:, pl.ds(0, IDX_W)], isem[p])

        def wait_in(p):
            pltpu.make_async_copy(
                tt_hbm.at[:, pl.ds(0, IDX_W)],
                inbuf.at[p].at[:, pl.ds(0, IDX_W)], isem[p]).wait()

        def wait_ob(p):
            pltpu.make_async_copy(
                out_hbm.at[pl.ds(0, D), :], obuf.at[p], osem[p]).wait()

        def transpose(src, dst, ncols):
            @plsc.parallel_loop(0, ncols, unroll=4)
            def tp(c):
                cvec = jnp.full((L,), 0, jnp.int32) + c
                for dh in range(D // L):
                    v = plsc.load_gather(src, [dlane[dh], cvec])
                    q = c * D + dh * L
                    dst[q // IDX_W, pl.ds(q % IDX_W, L)] = v

        for p in range(2):
            pltpu.async_copy(
                obuf.at[p], out_hbm.at[pl.ds(grp(p) * D, D), :], osem[p])
            fire_in(p, p)

        def outer(t, carry):
            for sub in range(2):
                i_ = 2 * t + sub
                p = sub
                wait_in(p)
                wait_ob(p)
                transpose(inbuf.at[p], obuf.at[p], IDX_W)
                pltpu.async_copy(
                    obuf.at[p], out_hbm.at[pl.ds(grp(i_) * D, D), :],
                    osem[p])
                fire_in(i_ + 2, p)
            return carry

        lax.fori_loop(0, NGI // 2, outer, 0)

        for p in range(2):
            wait_in(p)
            wait_ob(p)

    return body


def kernel(x, table):
    B0, B1 = x.shape
    V, D = table.shape
    xt_flat = jnp.transpose(x).reshape(-1).astype(jnp.int32)
    t_rm = _table_transpose_kernel(V, D)(jnp.transpose(table))
    vfull = (V // IDX_W) * IDX_W
    tail = jnp.reshape(table[vfull:, :], ((V - vfull) * D // IDX_W, IDX_W))
    t_rm = lax.dynamic_update_slice(t_rm, tail, (vfull * D // IDX_W, 0))
    out = _lookup_kernel(B0, B1, V, D)(xt_flat, t_rm.reshape(V, D))
    return jnp.transpose(out, (2, 0, 1))
